# Initial kernel scaffold; baseline (speedup 1.0000x reference)
import jax
import jax.numpy as jnp
from jax.experimental import pallas as pl


def kernel(x, edge_index, params):
    raise NotImplementedError("write your pallas kernel here")


# speedup vs baseline: 27.4820x; 27.4820x over previous
import functools

import jax
import jax.numpy as jnp
from jax import lax
from jax.experimental import pallas as pl
from jax.experimental.pallas import tpu as pltpu
from jax.experimental.pallas import tpu_sc as plsc

_DIMS = [(128, 128), (128, 128), (128, 64), (64, 64), (64, 32), (32, 32),
         (32, 16), (16, 16), (16, 8), (8, 8), (8, 8), (8, 8), (8, 4),
         (4, 4), (4, 1)]
N = 10000
N_PAD = 10240
ROWS_PT = N_PAD // 16
NC, NS, LANES = 2, 16, 16
NW = NC * NS
BLK = 128
E_TOT = 320000 + N
NBLK = -(-E_TOT // (NW * BLK))
E_TILE = NBLK * BLK
E_PAD = NW * E_TILE


def _act(x, kind):
    if kind == "relu":
        return jnp.maximum(x, 0.0)
    if kind == "elu":
        return jnp.where(x > 0, x, jnp.exp(jnp.minimum(x, 0.0)) - 1.0)
    return x


def _attn_outputs(h, as_ref, ad_ref, h_ref, asr_ref, adt_ref, amax_ref,
                  dout, dout_p):
    asr = jnp.sum(h * as_ref[...], axis=1, keepdims=True)
    adt = jnp.sum(h * ad_ref[...], axis=1, keepdims=True)
    if dout_p > dout:
        h = jnp.concatenate(
            [h, jnp.zeros((N, dout_p - dout), jnp.float32)], axis=1)
    h_ref[...] = h
    asr_ref[...] = asr
    adt_ref[...] = adt
    amax_ref[...] = jnp.full((1, LANES), jnp.max(asr), jnp.float32)


def _tc_first(x, W, a_s, a_d, dout, dout_p):
    def body(x_ref, w_ref, as_ref, ad_ref, h_ref, asr_ref, adt_ref, amax_ref):
        h = jnp.dot(x_ref[...], w_ref[...],
                    preferred_element_type=jnp.float32)
        _attn_outputs(h, as_ref, ad_ref, h_ref, asr_ref, adt_ref, amax_ref,
                      dout, dout_p)

    return pl.pallas_call(
        body,
        out_shape=(
            jax.ShapeDtypeStruct((N, dout_p), jnp.float32),
            jax.ShapeDtypeStruct((N, 1), jnp.float32),
            jax.ShapeDtypeStruct((N, 1), jnp.float32),
            jax.ShapeDtypeStruct((1, LANES), jnp.float32),
        ),
    )(x, W, a_s, a_d)


def _tc_mid(acc, den3, bprev, W, a_s, a_d, dprev, dpp, act_kind, dout, dout_p):
    def body(acc_ref, den_ref, b_ref, w_ref, as_ref, ad_ref,
             h_ref, asr_ref, adt_ref, amax_ref):
        accs = acc_ref[0, :N, :dprev] + acc_ref[1, :N, :dprev]
        dens = den_ref[0, :N] + den_ref[1, :N]
        xin = accs / (dens + 1e-16) + b_ref[...]
        xin = _act(xin, act_kind)
        h = jnp.dot(xin, w_ref[...], preferred_element_type=jnp.float32)
        _attn_outputs(h, as_ref, ad_ref, h_ref, asr_ref, adt_ref, amax_ref,
                      dout, dout_p)

    return pl.pallas_call(
        body,
        out_shape=(
            jax.ShapeDtypeStruct((N, dout_p), jnp.float32),
            jax.ShapeDtypeStruct((N, 1), jnp.float32),
            jax.ShapeDtypeStruct((N, 1), jnp.float32),
            jax.ShapeDtypeStruct((1, LANES), jnp.float32),
        ),
    )(acc, den3, bprev, W, a_s, a_d)


def _tc_final(acc, den3, b, dprev):
    def body(acc_ref, den_ref, b_ref, o_ref):
        accs = acc_ref[0, :N, :dprev] + acc_ref[1, :N, :dprev]
        dens = den_ref[0, :N] + den_ref[1, :N]
        o_ref[...] = accs / (dens + 1e-16) + b_ref[...]

    return pl.pallas_call(
        body,
        out_shape=jax.ShapeDtypeStruct((N, dprev), jnp.float32),
    )(acc, den3, b)


@functools.lru_cache(maxsize=None)
def _sc_edge(dout_p):
    CH = dout_p // LANES
    mesh = plsc.VectorSubcoreMesh(core_axis_name="c", subcore_axis_name="s")

    def body(srcb, dstb, h_hbm, asr_hbm, adt_hbm, amax_hbm,
             acc_hbm, den_hbm,
             src_v, dst_v, amax_v, w_v, sa_v, da_v, rows_v,
             asr_sh, adt_sh, acc_sh, den_sh):
        c = lax.axis_index("c")
        s = lax.axis_index("s")
        wid = c * NS + s

        pltpu.sync_copy(srcb.at[wid], src_v)
        pltpu.sync_copy(dstb.at[wid], dst_v)
        pltpu.sync_copy(amax_hbm, amax_v)

        @pl.when(s == 0)
        def _():
            pltpu.sync_copy(asr_hbm, asr_sh)
            pltpu.sync_copy(adt_hbm, adt_sh)

        z16 = jnp.zeros((LANES,), jnp.float32)

        def zrow(r, carry):
            for ch in range(CH):
                rows_v[r, pl.ds(ch * LANES, LANES)] = z16
            return carry

        lax.fori_loop(0, BLK, zrow, 0)
        for k in range(BLK // LANES):
            w_v[pl.ds(k * LANES, LANES)] = z16

        r0 = s * ROWS_PT
        for off in range(0, ROWS_PT, BLK):
            nn = min(BLK, ROWS_PT - off)
            pltpu.sync_copy(rows_v.at[pl.ds(0, nn)],
                            acc_sh.at[pl.ds(r0 + off, nn)])
            pltpu.sync_copy(w_v.at[pl.ds(0, nn)],
                            den_sh.at[pl.ds(r0 + off, nn)])
        plsc.subcore_barrier()

        def rowf(g, carry):
            w16 = w_v[pl.ds(g * LANES, LANES)]
            for l in range(LANES):
                wv = jnp.full((LANES,), w16[l], jnp.float32)
                r = g * LANES + l
                for ch in range(CH):
                    sl = pl.ds(ch * LANES, LANES)
                    rows_v[r, sl] = rows_v[r, sl] * wv
            return carry

        def blk_body(j, amax16):
            pltpu.sync_copy(h_hbm.at[src_v.at[j]], rows_v)
            pltpu.sync_copy(asr_sh.at[src_v.at[j]], sa_v)
            pltpu.sync_copy(adt_sh.at[dst_v.at[j]], da_v)
            for k in range(BLK // LANES):
                sa = sa_v[pl.ds(k * LANES, LANES)]
                da = da_v[pl.ds(k * LANES, LANES)]
                t = sa + da
                e = jnp.where(t >= 0, t, t * 0.2)
                m0 = amax16 + da
                m = jnp.where(m0 >= 0, m0, m0 * 0.2)
                eid = (wid * E_TILE + j * BLK + k * LANES) \
                    + lax.iota(jnp.int32, 16)
                w = jnp.where(eid < E_TOT, jnp.exp(e - m), 0.0)
                w_v[pl.ds(k * LANES, LANES)] = w
            lax.fori_loop(0, BLK // LANES, rowf, 0)
            pltpu.sync_copy(rows_v, acc_sh.at[dst_v.at[j]], add=True)
            pltpu.sync_copy(w_v, den_sh.at[dst_v.at[j]], add=True)
            return amax16

        lax.fori_loop(0, NBLK, blk_body, amax_v[...])

        plsc.subcore_barrier()
        pltpu.sync_copy(acc_sh.at[pl.ds(r0, ROWS_PT)],
                        acc_hbm.at[c, pl.ds(r0, ROWS_PT)])
        pltpu.sync_copy(den_sh.at[pl.ds(r0, ROWS_PT)],
                        den_hbm.at[c, pl.ds(r0, ROWS_PT)])

    return pl.kernel(
        body,
        out_type=(
            jax.ShapeDtypeStruct((NC, N_PAD, dout_p), jnp.float32),
            jax.ShapeDtypeStruct((NC, N_PAD), jnp.float32),
        ),
        mesh=mesh,
        compiler_params=pltpu.CompilerParams(needs_layout_passes=False),
        scratch_types=[
            pltpu.VMEM((NBLK, BLK), jnp.int32),
            pltpu.VMEM((NBLK, BLK), jnp.int32),
            pltpu.VMEM((LANES,), jnp.float32),
            pltpu.VMEM((BLK,), jnp.float32),
            pltpu.VMEM((BLK,), jnp.float32),
            pltpu.VMEM((BLK,), jnp.float32),
            pltpu.VMEM((BLK, dout_p), jnp.float32),
            pltpu.VMEM_SHARED((N,), jnp.float32),
            pltpu.VMEM_SHARED((N,), jnp.float32),
            pltpu.VMEM_SHARED((N_PAD, dout_p), jnp.float32),
            pltpu.VMEM_SHARED((N_PAD,), jnp.float32),
        ],
    )


def kernel(x, edge_index, params):
    ei = edge_index.astype(jnp.int32)
    loop = jnp.arange(N, dtype=jnp.int32)
    src = jnp.concatenate([ei[0], loop])
    dst = jnp.concatenate([ei[1], loop])
    srcb = jnp.pad(src, (0, E_PAD - E_TOT)).reshape(NW, NBLK, BLK)
    dstb = jnp.pad(dst, (0, E_PAD - E_TOT)).reshape(NW, NBLK, BLK)

    acc = den3 = None
    for i, (din, dout) in enumerate(_DIMS):
        dout_p = 128
        W, a_s, a_d, _ = params[i]
        if i == 0:
            hp, asr, adt, amax = _tc_first(x, W, a_s, a_d, dout, dout_p)
        else:
            dprev = _DIMS[i - 1][1]
            dpp = max(LANES, dprev)
            bprev = params[i - 1][3].reshape(1, dprev)
            act_kind = "elu" if (i - 1) in (8, 9) else "relu"
            hp, asr, adt, amax = _tc_mid(acc, den3, bprev, W, a_s, a_d,
                                         dprev, dpp, act_kind, dout, dout_p)
        acc, den = _sc_edge(dout_p)(
            srcb, dstb, hp, asr.reshape(N), adt.reshape(N),
            amax.reshape(LANES))
        den3 = den.reshape(NC, N_PAD, 1)

    return _tc_final(acc, den3, params[-1][3].reshape(1, 1), _DIMS[-1][1])


# speedup vs baseline: 29.9453x; 1.0896x over previous
import functools

import jax
import jax.numpy as jnp
from jax import lax
from jax.experimental import pallas as pl
from jax.experimental.pallas import tpu as pltpu
from jax.experimental.pallas import tpu_sc as plsc

_DIMS = [(128, 128), (128, 128), (128, 64), (64, 64), (64, 32), (32, 32),
         (32, 16), (16, 16), (16, 8), (8, 8), (8, 8), (8, 8), (8, 4),
         (4, 4), (4, 1)]
N = 10000
N_PAD = 10240
ROWS_PT = N_PAD // 16
NC, NS, LANES = 2, 16, 16
NW = NC * NS
BLK = 128
E_TOT = 320000 + N
NBLK = -(-E_TOT // (NW * BLK))
E_TILE = NBLK * BLK
E_PAD = NW * E_TILE


def _act(x, kind):
    if kind == "relu":
        return jnp.maximum(x, 0.0)
    if kind == "elu":
        return jnp.where(x > 0, x, jnp.exp(jnp.minimum(x, 0.0)) - 1.0)
    return x


def _attn_outputs(h, as_ref, ad_ref, h_ref, asr_ref, adt_ref, amax_ref,
                  dout, dout_p):
    asr = jnp.sum(h * as_ref[...], axis=1, keepdims=True)
    adt = jnp.sum(h * ad_ref[...], axis=1, keepdims=True)
    if dout_p > dout:
        h = jnp.concatenate(
            [h, jnp.zeros((N, dout_p - dout), jnp.float32)], axis=1)
    h_ref[...] = h
    asr_ref[...] = asr
    adt_ref[...] = adt
    amax_ref[...] = jnp.full((1, LANES), jnp.max(asr), jnp.float32)


def _tc_first(x, W, a_s, a_d, dout, dout_p):
    def body(x_ref, w_ref, as_ref, ad_ref, h_ref, asr_ref, adt_ref, amax_ref):
        h = jnp.dot(x_ref[...], w_ref[...],
                    preferred_element_type=jnp.float32)
        _attn_outputs(h, as_ref, ad_ref, h_ref, asr_ref, adt_ref, amax_ref,
                      dout, dout_p)

    return pl.pallas_call(
        body,
        out_shape=(
            jax.ShapeDtypeStruct((N, dout_p), jnp.float32),
            jax.ShapeDtypeStruct((N, 1), jnp.float32),
            jax.ShapeDtypeStruct((N, 1), jnp.float32),
            jax.ShapeDtypeStruct((1, LANES), jnp.float32),
        ),
    )(x, W, a_s, a_d)


def _tc_mid(acc, den3, bprev, W, a_s, a_d, dprev, dpp, act_kind, dout, dout_p):
    def body(acc_ref, den_ref, b_ref, w_ref, as_ref, ad_ref,
             h_ref, asr_ref, adt_ref, amax_ref):
        accs = acc_ref[0, :N, :dprev] + acc_ref[1, :N, :dprev]
        dens = den_ref[0, :N] + den_ref[1, :N]
        xin = accs / (dens + 1e-16) + b_ref[...]
        xin = _act(xin, act_kind)
        h = jnp.dot(xin, w_ref[...], preferred_element_type=jnp.float32)
        _attn_outputs(h, as_ref, ad_ref, h_ref, asr_ref, adt_ref, amax_ref,
                      dout, dout_p)

    return pl.pallas_call(
        body,
        out_shape=(
            jax.ShapeDtypeStruct((N, dout_p), jnp.float32),
            jax.ShapeDtypeStruct((N, 1), jnp.float32),
            jax.ShapeDtypeStruct((N, 1), jnp.float32),
            jax.ShapeDtypeStruct((1, LANES), jnp.float32),
        ),
    )(acc, den3, bprev, W, a_s, a_d)


def _tc_final(acc, den3, b, dprev):
    def body(acc_ref, den_ref, b_ref, o_ref):
        accs = acc_ref[0, :N, :dprev] + acc_ref[1, :N, :dprev]
        dens = den_ref[0, :N] + den_ref[1, :N]
        o_ref[...] = accs / (dens + 1e-16) + b_ref[...]

    return pl.pallas_call(
        body,
        out_shape=jax.ShapeDtypeStruct((N, dprev), jnp.float32),
    )(acc, den3, b)


@functools.lru_cache(maxsize=None)
def _sc_edge(dout_sc):
    CH = dout_sc // LANES
    mesh = plsc.VectorSubcoreMesh(core_axis_name="c", subcore_axis_name="s")

    def body(srcb, dstb, h_hbm, asr_hbm, adt_hbm, amax_hbm,
             acc_hbm, den_hbm,
             src_v, dst_v, amax_v, w_v, sa_v, da_v, rows_g,
             asr_sh, adt_sh, acc_sh, den_sh):
        rows_s = rows_g
        c = lax.axis_index("c")
        s = lax.axis_index("s")
        wid = c * NS + s

        pltpu.sync_copy(srcb.at[wid], src_v)
        pltpu.sync_copy(dstb.at[wid], dst_v)
        pltpu.sync_copy(amax_hbm, amax_v)

        @pl.when(s == 0)
        def _():
            pltpu.sync_copy(asr_hbm, asr_sh)
            pltpu.sync_copy(adt_hbm, adt_sh)

        z16 = jnp.zeros((LANES,), jnp.float32)

        def zrow(r, carry):
            for ch in range(128 // LANES):
                rows_s[r, pl.ds(ch * LANES, LANES)] = z16
            return carry

        lax.fori_loop(0, BLK, zrow, 0)
        for k in range(BLK // LANES):
            w_v[pl.ds(k * LANES, LANES)] = z16

        r0 = s * ROWS_PT
        for off in range(0, ROWS_PT, BLK):
            nn = min(BLK, ROWS_PT - off)
            pltpu.sync_copy(rows_s.at[pl.ds(0, nn)],
                            acc_sh.at[pl.ds(r0 + off, nn)])
            pltpu.sync_copy(w_v.at[pl.ds(0, nn)],
                            den_sh.at[pl.ds(r0 + off, nn)])
        plsc.subcore_barrier()

        def rowf(g, carry):
            w16 = w_v[pl.ds(g * LANES, LANES)]
            for l in range(LANES):
                wv = jnp.full((LANES,), w16[l], jnp.float32)
                r = g * LANES + l
                for ch in range(CH):
                    sl = pl.ds(ch * LANES, LANES)
                    rows_s[r, sl] = rows_g[r, sl] * wv
            return carry

        def blk_body(j, amax16):
            pltpu.sync_copy(h_hbm.at[src_v.at[j]], rows_g)
            pltpu.sync_copy(asr_sh.at[src_v.at[j]], sa_v)
            pltpu.sync_copy(adt_sh.at[dst_v.at[j]], da_v)
            for k in range(BLK // LANES):
                sa = sa_v[pl.ds(k * LANES, LANES)]
                da = da_v[pl.ds(k * LANES, LANES)]
                t = sa + da
                e = jnp.where(t >= 0, t, t * 0.2)
                m0 = amax16 + da
                m = jnp.where(m0 >= 0, m0, m0 * 0.2)
                eid = (wid * E_TILE + j * BLK + k * LANES) \
                    + lax.iota(jnp.int32, 16)
                w = jnp.where(eid < E_TOT, jnp.exp(e - m), 0.0)
                w_v[pl.ds(k * LANES, LANES)] = w
            lax.fori_loop(0, BLK // LANES, rowf, 0)
            pltpu.sync_copy(rows_s, acc_sh.at[dst_v.at[j]], add=True)
            pltpu.sync_copy(w_v, den_sh.at[dst_v.at[j]], add=True)
            return amax16

        lax.fori_loop(0, NBLK, blk_body, amax_v[...])

        plsc.subcore_barrier()
        pltpu.sync_copy(acc_sh.at[pl.ds(r0, ROWS_PT)],
                        acc_hbm.at[c, pl.ds(r0, ROWS_PT)])
        pltpu.sync_copy(den_sh.at[pl.ds(r0, ROWS_PT)],
                        den_hbm.at[c, pl.ds(r0, ROWS_PT)])

    return pl.kernel(
        body,
        out_type=(
            jax.ShapeDtypeStruct((NC, N_PAD, 128), jnp.float32),
            jax.ShapeDtypeStruct((NC, N_PAD), jnp.float32),
        ),
        mesh=mesh,
        compiler_params=pltpu.CompilerParams(needs_layout_passes=False),
        scratch_types=[
            pltpu.VMEM((NBLK, BLK), jnp.int32),
            pltpu.VMEM((NBLK, BLK), jnp.int32),
            pltpu.VMEM((LANES,), jnp.float32),
            pltpu.VMEM((BLK,), jnp.float32),
            pltpu.VMEM((BLK,), jnp.float32),
            pltpu.VMEM((BLK,), jnp.float32),
            pltpu.VMEM((BLK, 128), jnp.float32),
            pltpu.VMEM_SHARED((N,), jnp.float32),
            pltpu.VMEM_SHARED((N,), jnp.float32),
            pltpu.VMEM_SHARED((N_PAD, 128), jnp.float32),
            pltpu.VMEM_SHARED((N_PAD,), jnp.float32),
        ],
    )


def kernel(x, edge_index, params):
    ei = edge_index.astype(jnp.int32)
    loop = jnp.arange(N, dtype=jnp.int32)
    src = jnp.concatenate([ei[0], loop])
    dst = jnp.concatenate([ei[1], loop])
    srcb = jnp.pad(src, (0, E_PAD - E_TOT)).reshape(NW, NBLK, BLK)
    dstb = jnp.pad(dst, (0, E_PAD - E_TOT)).reshape(NW, NBLK, BLK)

    acc = den3 = None
    for i, (din, dout) in enumerate(_DIMS):
        dout_p = 128
        W, a_s, a_d, _ = params[i]
        if i == 0:
            hp, asr, adt, amax = _tc_first(x, W, a_s, a_d, dout, dout_p)
        else:
            dprev = _DIMS[i - 1][1]
            dpp = max(LANES, dprev)
            bprev = params[i - 1][3].reshape(1, dprev)
            act_kind = "elu" if (i - 1) in (8, 9) else "relu"
            hp, asr, adt, amax = _tc_mid(acc, den3, bprev, W, a_s, a_d,
                                         dprev, dpp, act_kind, dout, dout_p)
        dout_sc = max(LANES, dout)
        acc, den = _sc_edge(dout_sc)(
            srcb, dstb, hp, asr.reshape(N), adt.reshape(N),
            amax.reshape(LANES))
        den3 = den.reshape(NC, N_PAD, 1)

    return _tc_final(acc, den3, params[-1][3].reshape(1, 1), _DIMS[-1][1])
